# baseline (speedup 1.0000x reference)
import functools

import jax
import jax.numpy as jnp
from jax.experimental import pallas as pl
from jax.experimental.pallas import tpu as pltpu
from jax.experimental.pallas import tpu_sc as plsc

NH = 16

F32 = jnp.float32
BF16 = jnp.bfloat16


def _ln(x, g, b, eps=1e-5):
    m = jnp.mean(x, axis=-1, keepdims=True)
    v = jnp.mean((x - m) ** 2, axis=-1, keepdims=True)
    return (x - m) / jnp.sqrt(v + eps) * g + b


def _qkv_kernel(x_ref, g_ref, b_ref, w_ref, o_ref):
    h = _ln(x_ref[...], g_ref[...], b_ref[...])
    o_ref[...] = jnp.dot(h.astype(BF16), w_ref[...], preferred_element_type=F32)


def _attn_kernel(q_ref, k_ref, v_ref, o_ref, s_ref, *, blk, hd, hpb):
    qi = pl.program_id(1)
    scale = 1.0 / hd ** 0.5
    rows = jax.lax.broadcasted_iota(jnp.int32, (blk, blk), 0)
    cols = jax.lax.broadcasted_iota(jnp.int32, (blk, blk), 1)
    for p in range(hpb):
        sl = slice(p * hd, (p + 1) * hd)
        q = q_ref[:, sl].astype(BF16)

        def s_body(j, m, q=q, sl=sl):
            k = k_ref[pl.ds(j * blk, blk), sl].astype(BF16)
            s = jax.lax.dot_general(q, k, (((1,), (1,)), ((), ())),
                                    preferred_element_type=F32) * scale
            s = jnp.where((j < qi) | (rows >= cols), s, -jnp.inf)
            s_ref[:, pl.ds(j * blk, blk)] = s
            return jnp.maximum(m, jnp.max(s, axis=-1, keepdims=True))

        m = jax.lax.fori_loop(0, qi + 1, s_body,
                              jnp.full((blk, 1), -jnp.inf, F32))

        def l_body(j, l, m=m):
            p = jnp.exp(s_ref[:, pl.ds(j * blk, blk)] - m)
            s_ref[:, pl.ds(j * blk, blk)] = p
            return l + jnp.sum(p, axis=-1, keepdims=True)

        l = jax.lax.fori_loop(0, qi + 1, l_body, jnp.zeros((blk, 1), F32))
        inv = 1.0 / l

        def pv_body(j, acc, inv=inv, sl=sl):
            pb = (s_ref[:, pl.ds(j * blk, blk)] * inv).astype(BF16)
            v = v_ref[pl.ds(j * blk, blk), sl].astype(BF16)
            return acc + jnp.dot(pb, v, preferred_element_type=F32)

        acc = jax.lax.fori_loop(0, qi + 1, pv_body, jnp.zeros((blk, hd), F32))
        o_ref[:, sl] = acc


def _post_kernel(x_ref, y_ref, wp_ref, g_ref, b_ref, wr_ref,
                 x1_ref, h2_ref, comb_ref, i1_ref, i2_ref, wv1_ref, wv2_ref):
    y = jnp.dot(y_ref[...].astype(BF16), wp_ref[...], preferred_element_type=F32)
    x1 = x_ref[...] + y
    x1_ref[...] = x1
    h2 = _ln(x1, g_ref[...], b_ref[...])
    h2_ref[...] = h2
    logits = jnp.dot(h2.astype(BF16), wr_ref[...],
                     preferred_element_type=F32)
    n_e = logits.shape[-1]
    lane = jax.lax.broadcasted_iota(jnp.int32, logits.shape, 1)
    m1 = jnp.max(logits, axis=-1, keepdims=True)
    i1 = jnp.min(jnp.where(logits == m1, lane, n_e), axis=-1, keepdims=True)
    first1 = lane == i1
    rest = jnp.where(first1, -jnp.inf, logits)
    m2 = jnp.max(rest, axis=-1, keepdims=True)
    i2 = jnp.min(jnp.where(rest == m2, lane, n_e), axis=-1, keepdims=True)
    first2 = lane == i2
    t = jnp.exp(m2 - m1)
    w1 = 1.0 / (1.0 + t)
    w2 = t / (1.0 + t)
    comb_ref[...] = w1 * first1.astype(F32) + w2 * first2.astype(F32)
    i1_ref[...] = i1
    i2_ref[...] = i2
    wv1_ref[...] = w1
    wv2_ref[...] = w2


def _route_kernel(icat_ref, dest_ref, texp_ref, tvalid_ref,
                  csum_ref, coff_ref, *, n_e, blk, n_tiles, chunk):
    na = icat_ref.shape[0]
    n_chunks = na // chunk
    lane = jax.lax.broadcasted_iota(jnp.int32, (chunk, n_e), 1)
    ltri = (jax.lax.broadcasted_iota(jnp.int32, (chunk, chunk), 0) >
            jax.lax.broadcasted_iota(jnp.int32, (chunk, chunk), 1)).astype(F32)

    def sums_body(c, _):
        ids = icat_ref[pl.ds(c * chunk, chunk), :]
        oh = (lane == ids).astype(F32)
        csum_ref[pl.ds(c, 1), :] = jnp.sum(oh, axis=0, keepdims=True)
        return 0

    jax.lax.fori_loop(0, n_chunks, sums_body, 0)

    csum = csum_ref[...]
    ltri_c = (jax.lax.broadcasted_iota(jnp.int32, (n_chunks, n_chunks), 0) >
              jax.lax.broadcasted_iota(jnp.int32, (n_chunks, n_chunks), 1)).astype(F32)
    coff_ref[...] = jnp.dot(ltri_c, csum, preferred_element_type=F32)
    counts = jnp.sum(csum, axis=0, keepdims=True)
    cnt_pad = jnp.ceil(counts / blk) * blk
    utri_e = (jax.lax.broadcasted_iota(jnp.int32, (n_e, n_e), 0) <
              jax.lax.broadcasted_iota(jnp.int32, (n_e, n_e), 1)).astype(F32)
    off = jnp.dot(cnt_pad, utri_e, preferred_element_type=F32)
    total = jnp.sum(cnt_pad)

    def dest_body(c, _):
        ids = icat_ref[pl.ds(c * chunk, chunk), :]
        oh = (lane == ids).astype(F32)
        rank = jnp.dot(ltri, oh, preferred_element_type=F32) + coff_ref[pl.ds(c, 1), :]
        d = jnp.sum((off + rank) * oh, axis=1, keepdims=True)
        dest_ref[pl.ds(c * chunk, chunk), :] = d.astype(jnp.int32)
        return 0

    jax.lax.fori_loop(0, n_chunks, dest_body, 0)

    t_iota = jax.lax.broadcasted_iota(jnp.int32, (n_tiles, 1), 0)
    t_base = (t_iota * blk).astype(F32)
    texp = jnp.sum((t_base >= off).astype(jnp.int32), axis=1, keepdims=True) - 1
    texp_ref[...] = jnp.maximum(texp, 0)
    tvalid_ref[...] = (t_base < total).astype(jnp.int32)


def _moe_sparse_kernel(texp_sref, tvalid_sref,
                       h2_ref, d0r_ref, d1r_ref, w0r_ref, w1r_ref,
                       w1_ref, b1_ref, w2_ref, b2_ref,
                       oe_ref, xs_ref, acc_ref, *, blk, n_hblk):
    t = pl.program_id(0)
    h = pl.program_id(1)

    valid = tvalid_sref[t] == 1

    @pl.when(valid & (h == 0))
    def _():
        base = t * blk
        rcol = jax.lax.broadcasted_iota(jnp.int32, (blk, 1), 0) + base
        dsT = ((d0r_ref[...] == rcol).astype(BF16) +
               (d1r_ref[...] == rcol).astype(BF16))
        xs_ref[...] = jnp.dot(dsT, h2_ref[...], preferred_element_type=F32)
        acc_ref[...] = jnp.zeros_like(acc_ref)

    @pl.when(valid)
    def _():
        h1 = jnp.dot(xs_ref[...].astype(BF16), w1_ref[0],
                     preferred_element_type=F32) + b1_ref[0]
        h1 = h1 * 0.5 * (1.0 + jax.lax.erf(h1 * (2.0 ** -0.5)))
        acc_ref[...] += jnp.dot(h1.astype(BF16), w2_ref[0],
                                preferred_element_type=F32)

    @pl.when(valid & (h == n_hblk - 1))
    def _():
        base = t * blk
        rcol = jax.lax.broadcasted_iota(jnp.int32, (blk, 1), 0) + base
        wrow = jnp.sum(
            jnp.where(d0r_ref[...] == rcol, w0r_ref[...], 0.0) +
            jnp.where(d1r_ref[...] == rcol, w1r_ref[...], 0.0),
            axis=1, keepdims=True)
        oe_ref[...] = wrow * (acc_ref[...] + b2_ref[0])

    @pl.when(~valid)
    def _():
        oe_ref[...] = jnp.zeros_like(oe_ref)


def _sc_combine_body(oe_hbm, x1_hbm, d0_hbm, d1_hbm, out_hbm,
                     idx0_v, idx1_v, buf_v, r0_v, r1_v, sem,
                     *, bpw, n_cores, chunk, dim):
    wid = jax.lax.axis_index("s") * n_cores + jax.lax.axis_index("c")
    nj = dim // 16
    for c in range(bpw // chunk):
        base = wid * bpw + c * chunk
        pltpu.sync_copy(x1_hbm.at[pl.ds(base, chunk)], buf_v)
        pltpu.sync_copy(d0_hbm.at[pl.ds(base, chunk)], idx0_v)
        pltpu.sync_copy(d1_hbm.at[pl.ds(base, chunk)], idx1_v)
        cp0 = pltpu.async_copy(oe_hbm.at[idx0_v], r0_v, sem)
        cp1 = pltpu.async_copy(oe_hbm.at[idx1_v], r1_v, sem)
        cp0.wait()
        cp1.wait()
        for i in range(chunk):
            def addbody(j, _, i=i):
                sl = pl.ds(j * 16, 16)
                buf_v[i, sl] = buf_v[i, sl] + r0_v[i, sl] + r1_v[i, sl]
                return 0
            jax.lax.fori_loop(0, nj, addbody, 0)
        pltpu.sync_copy(buf_v, out_hbm.at[pl.ds(base, chunk)])


def kernel(x, ln1_g, ln1_b, W_attn, W_proj, ln2_g, ln2_b, W_router, W1, b1, W2, b2):
    Bb, T, DIM = x.shape
    E = W_router.shape[1]
    HID = W1.shape[2]
    hd = DIM // NH

    x2 = x.reshape(T, DIM)
    g1 = ln1_g.reshape(1, DIM)
    b1v = ln1_b.reshape(1, DIM)
    g2 = ln2_g.reshape(1, DIM)
    b2v = ln2_b.reshape(1, DIM)

    BT = min(256, T)
    nt = T // BT

    qkv = pl.pallas_call(
        _qkv_kernel,
        grid=(nt,),
        in_specs=[
            pl.BlockSpec((BT, DIM), lambda i: (i, 0)),
            pl.BlockSpec((1, DIM), lambda i: (0, 0)),
            pl.BlockSpec((1, DIM), lambda i: (0, 0)),
            pl.BlockSpec((DIM, 3 * DIM), lambda i: (0, 0)),
        ],
        out_specs=pl.BlockSpec((BT, 3 * DIM), lambda i: (i, 0)),
        out_shape=jax.ShapeDtypeStruct((T, 3 * DIM), F32),
    )(x2, g1, b1v, W_attn.astype(BF16))

    BQ = min(256, T)
    nq = T // BQ
    hpb = max(1, 128 // hd)
    WB = hpb * hd
    ng = NH // hpb
    nw = DIM // WB
    y = pl.pallas_call(
        functools.partial(_attn_kernel, blk=BQ, hd=hd, hpb=hpb),
        grid=(ng, nq),
        in_specs=[
            pl.BlockSpec((BQ, WB), lambda h, i: (i, h)),
            pl.BlockSpec((T, WB), lambda h, i: (0, nw + h)),
            pl.BlockSpec((T, WB), lambda h, i: (0, 2 * nw + h)),
        ],
        out_specs=pl.BlockSpec((BQ, WB), lambda h, i: (i, h)),
        out_shape=jax.ShapeDtypeStruct((T, DIM), F32),
        scratch_shapes=[pltpu.VMEM((BQ, T), F32)],
    )(qkv, qkv, qkv)

    x1, h2, comb, i1, i2, wv1, wv2 = pl.pallas_call(
        _post_kernel,
        grid=(nt,),
        in_specs=[
            pl.BlockSpec((BT, DIM), lambda i: (i, 0)),
            pl.BlockSpec((BT, DIM), lambda i: (i, 0)),
            pl.BlockSpec((DIM, DIM), lambda i: (0, 0)),
            pl.BlockSpec((1, DIM), lambda i: (0, 0)),
            pl.BlockSpec((1, DIM), lambda i: (0, 0)),
            pl.BlockSpec((DIM, E), lambda i: (0, 0)),
        ],
        out_specs=[
            pl.BlockSpec((BT, DIM), lambda i: (i, 0)),
            pl.BlockSpec((BT, DIM), lambda i: (i, 0)),
            pl.BlockSpec((BT, E), lambda i: (i, 0)),
            pl.BlockSpec((BT, 1), lambda i: (i, 0)),
            pl.BlockSpec((BT, 1), lambda i: (i, 0)),
            pl.BlockSpec((BT, 1), lambda i: (i, 0)),
            pl.BlockSpec((BT, 1), lambda i: (i, 0)),
        ],
        out_shape=[
            jax.ShapeDtypeStruct((T, DIM), F32),
            jax.ShapeDtypeStruct((T, DIM), F32),
            jax.ShapeDtypeStruct((T, E), F32),
            jax.ShapeDtypeStruct((T, 1), jnp.int32),
            jax.ShapeDtypeStruct((T, 1), jnp.int32),
            jax.ShapeDtypeStruct((T, 1), F32),
            jax.ShapeDtypeStruct((T, 1), F32),
        ],
    )(x2, y, W_proj.astype(BF16), g2, b2v, W_router.astype(BF16))

    KTOP = 2
    NA = KTOP * T
    BLK = 128
    CHUNK = 128
    n_tiles = (NA + E * BLK) // BLK
    icat = jnp.concatenate([i1, i2], axis=0)
    dest, texp, tvalid = pl.pallas_call(
        functools.partial(_route_kernel, n_e=E, blk=BLK, n_tiles=n_tiles,
                          chunk=CHUNK),
        grid=(1,),
        in_specs=[pl.BlockSpec((NA, 1), lambda i: (0, 0))],
        out_specs=[
            pl.BlockSpec((NA, 1), lambda i: (0, 0)),
            pl.BlockSpec((n_tiles, 1), lambda i: (0, 0)),
            pl.BlockSpec((n_tiles, 1), lambda i: (0, 0)),
        ],
        out_shape=[
            jax.ShapeDtypeStruct((NA, 1), jnp.int32),
            jax.ShapeDtypeStruct((n_tiles, 1), jnp.int32),
            jax.ShapeDtypeStruct((n_tiles, 1), jnp.int32),
        ],
        scratch_shapes=[
            pltpu.VMEM((NA // CHUNK, E), F32),
            pltpu.VMEM((NA // CHUNK, E), F32),
        ],
    )(icat)

    drows = dest.reshape(KTOP, T)
    d0r = drows[0:1, :]
    d1r = drows[1:2, :]
    w0r = wv1.reshape(1, T)
    w1r = wv2.reshape(1, T)

    BH = min(1024, HID)
    nh_blk = HID // BH
    NPAD = n_tiles * BLK
    grid_spec = pltpu.PrefetchScalarGridSpec(
        num_scalar_prefetch=2,
        grid=(n_tiles, nh_blk),
        in_specs=[
            pl.BlockSpec((T, DIM), lambda t, h, texp, tval: (0, 0)),
            pl.BlockSpec((1, T), lambda t, h, texp, tval: (0, 0)),
            pl.BlockSpec((1, T), lambda t, h, texp, tval: (0, 0)),
            pl.BlockSpec((1, T), lambda t, h, texp, tval: (0, 0)),
            pl.BlockSpec((1, T), lambda t, h, texp, tval: (0, 0)),
            pl.BlockSpec((1, DIM, BH), lambda t, h, texp, tval: (texp[t], 0, h)),
            pl.BlockSpec((1, 1, BH), lambda t, h, texp, tval: (texp[t], 0, h)),
            pl.BlockSpec((1, BH, DIM), lambda t, h, texp, tval: (texp[t], h, 0)),
            pl.BlockSpec((1, 1, DIM), lambda t, h, texp, tval: (texp[t], 0, 0)),
        ],
        out_specs=pl.BlockSpec((BLK, DIM), lambda t, h, texp, tval: (t, 0)),
        scratch_shapes=[
            pltpu.VMEM((BLK, DIM), F32),
            pltpu.VMEM((BLK, DIM), F32),
        ],
    )
    oe_w = pl.pallas_call(
        functools.partial(_moe_sparse_kernel, blk=BLK, n_hblk=nh_blk),
        grid_spec=grid_spec,
        out_shape=jax.ShapeDtypeStruct((NPAD, DIM), F32),
        compiler_params=pltpu.CompilerParams(
            dimension_semantics=("arbitrary", "arbitrary"),
        ),
    )(texp.reshape(n_tiles), tvalid.reshape(n_tiles),
      h2.astype(BF16), d0r, d1r, w0r, w1r,
      W1.astype(BF16), b1.reshape(E, 1, HID), W2.astype(BF16),
      b2.reshape(E, 1, DIM))

    n_sc, n_sub = 2, 16
    bpw = T // (n_sc * n_sub)
    chunk = min(32, bpw)
    mesh = plsc.VectorSubcoreMesh(core_axis_name="c", subcore_axis_name="s")
    sc_combine = pl.kernel(
        functools.partial(_sc_combine_body, bpw=bpw, n_cores=n_sc,
                          chunk=chunk, dim=DIM),
        mesh=mesh,
        out_type=jax.ShapeDtypeStruct((T, DIM), F32),
        scratch_types=[
            pltpu.VMEM((chunk,), jnp.int32),
            pltpu.VMEM((chunk,), jnp.int32),
            pltpu.VMEM((chunk, DIM), F32),
            pltpu.VMEM((chunk, DIM), F32),
            pltpu.VMEM((chunk, DIM), F32),
            pltpu.SemaphoreType.DMA,
        ],
    )
    out = sc_combine(oe_w, x1, drows[0], drows[1])

    return out.reshape(Bb, T, DIM)


# speedup vs baseline: 1.4578x; 1.4578x over previous
import functools

import jax
import jax.numpy as jnp
from jax.experimental import pallas as pl
from jax.experimental.pallas import tpu as pltpu
from jax.experimental.pallas import tpu_sc as plsc

NH = 16

F32 = jnp.float32
BF16 = jnp.bfloat16


def _ln(x, g, b, eps=1e-5):
    m = jnp.mean(x, axis=-1, keepdims=True)
    v = jnp.mean((x - m) ** 2, axis=-1, keepdims=True)
    return (x - m) / jnp.sqrt(v + eps) * g + b


def _qkv_kernel(x_ref, g_ref, b_ref, w_ref, o_ref):
    h = _ln(x_ref[...], g_ref[...], b_ref[...])
    o_ref[...] = jnp.dot(h.astype(BF16), w_ref[...], preferred_element_type=F32)


def _attn_kernel(q_ref, k_ref, v_ref, o_ref, s_ref, *, blk, hd, hpb):
    qi = pl.program_id(1)
    scale = 1.0 / hd ** 0.5
    rows = jax.lax.broadcasted_iota(jnp.int32, (blk, blk), 0)
    cols = jax.lax.broadcasted_iota(jnp.int32, (blk, blk), 1)
    for p in range(hpb):
        sl = slice(p * hd, (p + 1) * hd)
        q = q_ref[:, sl].astype(BF16)

        def s_body(j, m, q=q, sl=sl):
            k = k_ref[pl.ds(j * blk, blk), sl].astype(BF16)
            s = jax.lax.dot_general(q, k, (((1,), (1,)), ((), ())),
                                    preferred_element_type=F32) * scale
            s = jnp.where((j < qi) | (rows >= cols), s, -jnp.inf)
            s_ref[:, pl.ds(j * blk, blk)] = s
            return jnp.maximum(m, jnp.max(s, axis=-1, keepdims=True))

        m = jax.lax.fori_loop(0, qi + 1, s_body,
                              jnp.full((blk, 1), -jnp.inf, F32))

        def l_body(j, l, m=m):
            p = jnp.exp(s_ref[:, pl.ds(j * blk, blk)] - m)
            s_ref[:, pl.ds(j * blk, blk)] = p
            return l + jnp.sum(p, axis=-1, keepdims=True)

        l = jax.lax.fori_loop(0, qi + 1, l_body, jnp.zeros((blk, 1), F32))
        inv = 1.0 / l

        def pv_body(j, acc, inv=inv, sl=sl):
            pb = (s_ref[:, pl.ds(j * blk, blk)] * inv).astype(BF16)
            v = v_ref[pl.ds(j * blk, blk), sl].astype(BF16)
            return acc + jnp.dot(pb, v, preferred_element_type=F32)

        acc = jax.lax.fori_loop(0, qi + 1, pv_body, jnp.zeros((blk, hd), F32))
        o_ref[:, sl] = acc


def _post_kernel(x_ref, y_ref, wp_ref, g_ref, b_ref, wr_ref,
                 x1_ref, h2_ref, comb_ref, i1_ref, i2_ref, wv1_ref, wv2_ref):
    y = jnp.dot(y_ref[...].astype(BF16), wp_ref[...], preferred_element_type=F32)
    x1 = x_ref[...] + y
    x1_ref[...] = x1
    h2 = _ln(x1, g_ref[...], b_ref[...])
    h2_ref[...] = h2
    logits = jnp.dot(h2.astype(BF16), wr_ref[...],
                     preferred_element_type=F32)
    n_e = logits.shape[-1]
    lane = jax.lax.broadcasted_iota(jnp.int32, logits.shape, 1)
    m1 = jnp.max(logits, axis=-1, keepdims=True)
    i1 = jnp.min(jnp.where(logits == m1, lane, n_e), axis=-1, keepdims=True)
    first1 = lane == i1
    rest = jnp.where(first1, -jnp.inf, logits)
    m2 = jnp.max(rest, axis=-1, keepdims=True)
    i2 = jnp.min(jnp.where(rest == m2, lane, n_e), axis=-1, keepdims=True)
    first2 = lane == i2
    t = jnp.exp(m2 - m1)
    w1 = 1.0 / (1.0 + t)
    w2 = t / (1.0 + t)
    comb_ref[...] = w1 * first1.astype(F32) + w2 * first2.astype(F32)
    i1_ref[...] = i1
    i2_ref[...] = i2
    wv1_ref[...] = w1
    wv2_ref[...] = w2


def _route_kernel(icat_ref, dest_ref, texp_ref, tvalid_ref,
                  csum_ref, coff_ref, *, n_e, blk, n_tiles, chunk):
    na = icat_ref.shape[0]
    n_chunks = na // chunk
    lane = jax.lax.broadcasted_iota(jnp.int32, (chunk, n_e), 1)
    ltri = (jax.lax.broadcasted_iota(jnp.int32, (chunk, chunk), 0) >
            jax.lax.broadcasted_iota(jnp.int32, (chunk, chunk), 1)).astype(F32)

    def sums_body(c, _):
        ids = icat_ref[pl.ds(c * chunk, chunk), :]
        oh = (lane == ids).astype(F32)
        csum_ref[pl.ds(c, 1), :] = jnp.sum(oh, axis=0, keepdims=True)
        return 0

    jax.lax.fori_loop(0, n_chunks, sums_body, 0)

    csum = csum_ref[...]
    ltri_c = (jax.lax.broadcasted_iota(jnp.int32, (n_chunks, n_chunks), 0) >
              jax.lax.broadcasted_iota(jnp.int32, (n_chunks, n_chunks), 1)).astype(F32)
    coff_ref[...] = jnp.dot(ltri_c, csum, preferred_element_type=F32)
    counts = jnp.sum(csum, axis=0, keepdims=True)
    cnt_pad = jnp.ceil(counts / blk) * blk
    utri_e = (jax.lax.broadcasted_iota(jnp.int32, (n_e, n_e), 0) <
              jax.lax.broadcasted_iota(jnp.int32, (n_e, n_e), 1)).astype(F32)
    off = jnp.dot(cnt_pad, utri_e, preferred_element_type=F32)
    total = jnp.sum(cnt_pad)

    def dest_body(c, _):
        ids = icat_ref[pl.ds(c * chunk, chunk), :]
        oh = (lane == ids).astype(F32)
        rank = jnp.dot(ltri, oh, preferred_element_type=F32) + coff_ref[pl.ds(c, 1), :]
        d = jnp.sum((off + rank) * oh, axis=1, keepdims=True)
        dest_ref[pl.ds(c * chunk, chunk), :] = d.astype(jnp.int32)
        return 0

    jax.lax.fori_loop(0, n_chunks, dest_body, 0)

    t_iota = jax.lax.broadcasted_iota(jnp.int32, (n_tiles, 1), 0)
    t_base = (t_iota * blk).astype(F32)
    texp = jnp.sum((t_base >= off).astype(jnp.int32), axis=1, keepdims=True) - 1
    texp_ref[...] = jnp.maximum(texp, 0)
    tvalid_ref[...] = (t_base < total).astype(jnp.int32)


def _moe_half_kernel(texp_sref, tvalid_sref,
                     h2_ref, d0r_ref, d1r_ref, w0r_ref, w1r_ref,
                     w1_ref, b1_ref, w2_ref, b2_ref, *rest,
                     blk, final):
    if final:
        prev_ref, oe_ref = rest
    else:
        prev_ref = None
        (oe_ref,) = rest
    t = pl.program_id(0)
    valid = tvalid_sref[t] == 1

    @pl.when(valid)
    def _():
        base = t * blk
        rcol = jax.lax.broadcasted_iota(jnp.int32, (blk, 1), 0) + base
        dsT = ((d0r_ref[...] == rcol).astype(BF16) +
               (d1r_ref[...] == rcol).astype(BF16))
        xs = jnp.dot(dsT, h2_ref[...], preferred_element_type=F32)
        h1 = jnp.dot(xs.astype(BF16), w1_ref[0],
                     preferred_element_type=F32) + b1_ref[0]
        h1 = h1 * 0.5 * (1.0 + jax.lax.erf(h1 * (2.0 ** -0.5)))
        acc = jnp.dot(h1.astype(BF16), w2_ref[0], preferred_element_type=F32)
        if final:
            wrow = jnp.sum(
                jnp.where(d0r_ref[...] == rcol, w0r_ref[...], 0.0) +
                jnp.where(d1r_ref[...] == rcol, w1r_ref[...], 0.0),
                axis=1, keepdims=True)
            oe_ref[...] = wrow * (prev_ref[...] + acc + b2_ref[0])
        else:
            oe_ref[...] = acc

    if final:
        @pl.when(~valid)
        def _():
            oe_ref[...] = jnp.zeros_like(oe_ref)


def _sc_combine_body(oe_hbm, x1_hbm, d0_hbm, d1_hbm, out_hbm,
                     idx0_v, idx1_v, buf_v, r0_v, r1_v, sem,
                     *, bpw, n_cores, chunk, dim):
    wid = jax.lax.axis_index("s") * n_cores + jax.lax.axis_index("c")
    nj = dim // 16
    for c in range(bpw // chunk):
        base = wid * bpw + c * chunk
        pltpu.sync_copy(x1_hbm.at[pl.ds(base, chunk)], buf_v)
        pltpu.sync_copy(d0_hbm.at[pl.ds(base, chunk)], idx0_v)
        pltpu.sync_copy(d1_hbm.at[pl.ds(base, chunk)], idx1_v)
        cp0 = pltpu.async_copy(oe_hbm.at[idx0_v], r0_v, sem)
        cp1 = pltpu.async_copy(oe_hbm.at[idx1_v], r1_v, sem)
        cp0.wait()
        cp1.wait()
        for i in range(chunk):
            def addbody(j, _, i=i):
                sl = pl.ds(j * 16, 16)
                buf_v[i, sl] = buf_v[i, sl] + r0_v[i, sl] + r1_v[i, sl]
                return 0
            jax.lax.fori_loop(0, nj, addbody, 0)
        pltpu.sync_copy(buf_v, out_hbm.at[pl.ds(base, chunk)])


def kernel(x, ln1_g, ln1_b, W_attn, W_proj, ln2_g, ln2_b, W_router, W1, b1, W2, b2):
    Bb, T, DIM = x.shape
    E = W_router.shape[1]
    HID = W1.shape[2]
    hd = DIM // NH

    x2 = x.reshape(T, DIM)
    g1 = ln1_g.reshape(1, DIM)
    b1v = ln1_b.reshape(1, DIM)
    g2 = ln2_g.reshape(1, DIM)
    b2v = ln2_b.reshape(1, DIM)

    BT = min(256, T)
    nt = T // BT

    qkv = pl.pallas_call(
        _qkv_kernel,
        grid=(nt,),
        in_specs=[
            pl.BlockSpec((BT, DIM), lambda i: (i, 0)),
            pl.BlockSpec((1, DIM), lambda i: (0, 0)),
            pl.BlockSpec((1, DIM), lambda i: (0, 0)),
            pl.BlockSpec((DIM, 3 * DIM), lambda i: (0, 0)),
        ],
        out_specs=pl.BlockSpec((BT, 3 * DIM), lambda i: (i, 0)),
        out_shape=jax.ShapeDtypeStruct((T, 3 * DIM), F32),
    )(x2, g1, b1v, W_attn.astype(BF16))

    BQ = min(512, T)
    nq = T // BQ
    hpb = max(1, 128 // hd)
    WB = hpb * hd
    ng = NH // hpb
    nw = DIM // WB
    y = pl.pallas_call(
        functools.partial(_attn_kernel, blk=BQ, hd=hd, hpb=hpb),
        grid=(ng, nq),
        in_specs=[
            pl.BlockSpec((BQ, WB), lambda h, i: (i, h)),
            pl.BlockSpec((T, WB), lambda h, i: (0, nw + h)),
            pl.BlockSpec((T, WB), lambda h, i: (0, 2 * nw + h)),
        ],
        out_specs=pl.BlockSpec((BQ, WB), lambda h, i: (i, h)),
        out_shape=jax.ShapeDtypeStruct((T, DIM), F32),
        scratch_shapes=[pltpu.VMEM((BQ, T), F32)],
    )(qkv, qkv, qkv)

    x1, h2, comb, i1, i2, wv1, wv2 = pl.pallas_call(
        _post_kernel,
        grid=(nt,),
        in_specs=[
            pl.BlockSpec((BT, DIM), lambda i: (i, 0)),
            pl.BlockSpec((BT, DIM), lambda i: (i, 0)),
            pl.BlockSpec((DIM, DIM), lambda i: (0, 0)),
            pl.BlockSpec((1, DIM), lambda i: (0, 0)),
            pl.BlockSpec((1, DIM), lambda i: (0, 0)),
            pl.BlockSpec((DIM, E), lambda i: (0, 0)),
        ],
        out_specs=[
            pl.BlockSpec((BT, DIM), lambda i: (i, 0)),
            pl.BlockSpec((BT, DIM), lambda i: (i, 0)),
            pl.BlockSpec((BT, E), lambda i: (i, 0)),
            pl.BlockSpec((BT, 1), lambda i: (i, 0)),
            pl.BlockSpec((BT, 1), lambda i: (i, 0)),
            pl.BlockSpec((BT, 1), lambda i: (i, 0)),
            pl.BlockSpec((BT, 1), lambda i: (i, 0)),
        ],
        out_shape=[
            jax.ShapeDtypeStruct((T, DIM), F32),
            jax.ShapeDtypeStruct((T, DIM), F32),
            jax.ShapeDtypeStruct((T, E), F32),
            jax.ShapeDtypeStruct((T, 1), jnp.int32),
            jax.ShapeDtypeStruct((T, 1), jnp.int32),
            jax.ShapeDtypeStruct((T, 1), F32),
            jax.ShapeDtypeStruct((T, 1), F32),
        ],
    )(x2, y, W_proj.astype(BF16), g2, b2v, W_router.astype(BF16))

    KTOP = 2
    NA = KTOP * T
    BLK = 128
    CHUNK = 128
    n_tiles = (NA + E * BLK) // BLK
    icat = jnp.concatenate([i1, i2], axis=0)
    dest, texp, tvalid = pl.pallas_call(
        functools.partial(_route_kernel, n_e=E, blk=BLK, n_tiles=n_tiles,
                          chunk=CHUNK),
        grid=(1,),
        in_specs=[pl.BlockSpec((NA, 1), lambda i: (0, 0))],
        out_specs=[
            pl.BlockSpec((NA, 1), lambda i: (0, 0)),
            pl.BlockSpec((n_tiles, 1), lambda i: (0, 0)),
            pl.BlockSpec((n_tiles, 1), lambda i: (0, 0)),
        ],
        out_shape=[
            jax.ShapeDtypeStruct((NA, 1), jnp.int32),
            jax.ShapeDtypeStruct((n_tiles, 1), jnp.int32),
            jax.ShapeDtypeStruct((n_tiles, 1), jnp.int32),
        ],
        scratch_shapes=[
            pltpu.VMEM((NA // CHUNK, E), F32),
            pltpu.VMEM((NA // CHUNK, E), F32),
        ],
    )(icat)

    drows = dest.reshape(KTOP, T)
    d0r = drows[0:1, :]
    d1r = drows[1:2, :]
    w0r = wv1.reshape(1, T)
    w1r = wv2.reshape(1, T)

    NPAD = n_tiles * BLK
    BH = HID // 2
    texp_f = texp.reshape(n_tiles)
    tval_f = tvalid.reshape(n_tiles)
    h2b = h2.astype(BF16)
    w1b = W1.astype(BF16)
    w2b = W2.astype(BF16)
    b1r3 = b1.reshape(E, 1, HID)
    b2r3 = b2.reshape(E, 1, DIM)

    def _half_specs(hh, final):
        specs = [
            pl.BlockSpec((T, DIM), lambda t, texp, tval: (0, 0)),
            pl.BlockSpec((1, T), lambda t, texp, tval: (0, 0)),
            pl.BlockSpec((1, T), lambda t, texp, tval: (0, 0)),
            pl.BlockSpec((1, T), lambda t, texp, tval: (0, 0)),
            pl.BlockSpec((1, T), lambda t, texp, tval: (0, 0)),
            pl.BlockSpec((1, DIM, BH), lambda t, texp, tval: (texp[t], 0, hh)),
            pl.BlockSpec((1, 1, BH), lambda t, texp, tval: (texp[t], 0, hh)),
            pl.BlockSpec((1, BH, DIM), lambda t, texp, tval: (texp[t], hh, 0)),
            pl.BlockSpec((1, 1, DIM), lambda t, texp, tval: (texp[t], 0, 0)),
        ]
        if final:
            specs.append(pl.BlockSpec((BLK, DIM), lambda t, texp, tval: (t, 0)))
        return specs

    oe_part = pl.pallas_call(
        functools.partial(_moe_half_kernel, blk=BLK, final=False),
        grid_spec=pltpu.PrefetchScalarGridSpec(
            num_scalar_prefetch=2,
            grid=(n_tiles,),
            in_specs=_half_specs(0, False),
            out_specs=pl.BlockSpec((BLK, DIM), lambda t, texp, tval: (t, 0)),
        ),
        out_shape=jax.ShapeDtypeStruct((NPAD, DIM), F32),
        compiler_params=pltpu.CompilerParams(
            dimension_semantics=("arbitrary",),
        ),
    )(texp_f, tval_f, h2b, d0r, d1r, w0r, w1r,
      w1b, b1r3, w2b, b2r3)

    oe_w = pl.pallas_call(
        functools.partial(_moe_half_kernel, blk=BLK, final=True),
        grid_spec=pltpu.PrefetchScalarGridSpec(
            num_scalar_prefetch=2,
            grid=(n_tiles,),
            in_specs=_half_specs(1, True),
            out_specs=pl.BlockSpec((BLK, DIM), lambda t, texp, tval: (t, 0)),
        ),
        out_shape=jax.ShapeDtypeStruct((NPAD, DIM), F32),
        compiler_params=pltpu.CompilerParams(
            dimension_semantics=("arbitrary",),
        ),
    )(texp_f, tval_f, h2b, d0r, d1r, w0r, w1r,
      w1b, b1r3, w2b, b2r3, oe_part)

    n_sc, n_sub = 2, 16
    bpw = T // (n_sc * n_sub)
    chunk = min(32, bpw)
    mesh = plsc.VectorSubcoreMesh(core_axis_name="c", subcore_axis_name="s")
    sc_combine = pl.kernel(
        functools.partial(_sc_combine_body, bpw=bpw, n_cores=n_sc,
                          chunk=chunk, dim=DIM),
        mesh=mesh,
        out_type=jax.ShapeDtypeStruct((T, DIM), F32),
        scratch_types=[
            pltpu.VMEM((chunk,), jnp.int32),
            pltpu.VMEM((chunk,), jnp.int32),
            pltpu.VMEM((chunk, DIM), F32),
            pltpu.VMEM((chunk, DIM), F32),
            pltpu.VMEM((chunk, DIM), F32),
            pltpu.SemaphoreType.DMA,
        ],
    )
    out = sc_combine(oe_w, x1, drows[0], drows[1])

    return out.reshape(Bb, T, DIM)
